# Initial kernel scaffold; baseline (speedup 1.0000x reference)
import jax
import jax.numpy as jnp
from jax.experimental import pallas as pl


def kernel(input, Wr, W1, b1, W2, b2):
    raise NotImplementedError("write your pallas kernel here")


# speedup vs baseline: 1.9298x; 1.9298x over previous
import functools

import jax
import jax.numpy as jnp
from jax import lax
from jax.experimental import pallas as pl
from jax.experimental.pallas import tpu as pltpu
from jax.experimental.pallas import tpu_sc as plsc

T = 4096
D = 768
E = 8
K = 2
F = 3072
C = 1280

TB = 512
NB = T // TB
FB = 768
NF = F // FB
NW = 32
SPW = (E * C) // NW
GCH = 64
TPW = T // NW


def _router_body(x_ref, wr_ref, t1_ref, t2_ref, g1_ref, g2_ref, loss_ref,
                 carry_ref, me_ref):
    pid = pl.program_id(0)

    @pl.when(pid == 0)
    def _init():
        carry_ref[...] = jnp.zeros_like(carry_ref)
        me_ref[...] = jnp.zeros_like(me_ref)

    x = x_ref[...]
    logits = jnp.dot(x, wr_ref[...], preferred_element_type=jnp.float32)
    m = jnp.max(logits, axis=1, keepdims=True)
    ex = jnp.exp(logits - m)
    probs = ex / jnp.sum(ex, axis=1, keepdims=True)

    eidx = lax.broadcasted_iota(jnp.int32, (TB, E), 1)
    v1 = jnp.max(probs, axis=1, keepdims=True)
    i1 = jnp.min(jnp.where(probs == v1, eidx, E), axis=1, keepdims=True)
    masked = jnp.where(eidx == i1, -jnp.inf, probs)
    v2 = jnp.max(masked, axis=1, keepdims=True)
    i2 = jnp.min(jnp.where(masked == v2, eidx, E), axis=1, keepdims=True)
    denom = v1 + v2
    g1 = v1 / denom
    g2 = v2 / denom

    oh1 = (eidx == i1).astype(jnp.float32)
    oh2 = (eidx == i2).astype(jnp.float32)
    cnt = oh1 + oh2
    r = lax.broadcasted_iota(jnp.int32, (TB, TB), 0)
    c = lax.broadcasted_iota(jnp.int32, (TB, TB), 1)
    ltri = (c < r).astype(jnp.float32)
    s = jnp.dot(ltri, cnt, preferred_element_type=jnp.float32)
    base = carry_ref[...] + s
    pos1 = jnp.sum(base * oh1, axis=1, keepdims=True)
    pos2 = jnp.sum((base + oh1) * oh2, axis=1, keepdims=True)
    carry_ref[...] = carry_ref[...] + jnp.sum(cnt, axis=0, keepdims=True)
    me_ref[...] = me_ref[...] + jnp.sum(probs, axis=0, keepdims=True)

    keep1 = pos1 < C
    keep2 = pos2 < C
    p1 = pos1.astype(jnp.int32)
    p2 = pos2.astype(jnp.int32)
    t1_ref[...] = jnp.where(keep1, i1 * C + p1, -1)
    t2_ref[...] = jnp.where(keep2, i2 * C + p2, -1)
    g1_ref[...] = g1 * keep1.astype(jnp.float32)
    g2_ref[...] = g2 * keep2.astype(jnp.float32)

    @pl.when(pid == NB - 1)
    def _loss():
        me = me_ref[...] / T
        ce = carry_ref[...] / (T * K)
        loss_ref[...] = E * jnp.sum(me * ce, keepdims=True)


def _router(x, wr):
    return pl.pallas_call(
        _router_body,
        grid=(NB,),
        in_specs=[
            pl.BlockSpec((TB, D), lambda i: (i, 0)),
            pl.BlockSpec((D, E), lambda i: (0, 0)),
        ],
        out_specs=[
            pl.BlockSpec((TB, 1), lambda i: (i, 0)),
            pl.BlockSpec((TB, 1), lambda i: (i, 0)),
            pl.BlockSpec((TB, 1), lambda i: (i, 0)),
            pl.BlockSpec((TB, 1), lambda i: (i, 0)),
            pl.BlockSpec((1, 1), lambda i: (0, 0)),
        ],
        out_shape=[
            jax.ShapeDtypeStruct((T, 1), jnp.int32),
            jax.ShapeDtypeStruct((T, 1), jnp.int32),
            jax.ShapeDtypeStruct((T, 1), jnp.float32),
            jax.ShapeDtypeStruct((T, 1), jnp.float32),
            jax.ShapeDtypeStruct((1, 1), jnp.float32),
        ],
        scratch_shapes=[
            pltpu.VMEM((1, E), jnp.float32),
            pltpu.VMEM((1, E), jnp.float32),
        ],
        compiler_params=pltpu.CompilerParams(
            dimension_semantics=("arbitrary",)),
    )(x, wr)


def _dispatch_body(x_hbm, t1_hbm, t2_hbm, g1_hbm, g2_hbm, buf_hbm, sg_hbm,
                   tgtv, gmv, tok, gat, rows, sem):
    wid = lax.axis_index("s") * 2 + lax.axis_index("c")
    lo = wid * SPW

    zi = jnp.zeros((16,), jnp.int32)
    zf = jnp.zeros((16,), jnp.float32)
    for j in range(SPW // 16):
        tok[pl.ds(j * 16, 16)] = zi
        gat[pl.ds(j * 16, 16)] = zf

    def scatter_pass(t_hbm, g_hbm):
        pltpu.sync_copy(t_hbm, tgtv)
        pltpu.sync_copy(g_hbm, gmv)

        def body(i, _):
            v = tgtv[pl.ds(i * 16, 16)]
            g = gmv[pl.ds(i * 16, 16)]
            tokval = i * 16 + lax.iota(jnp.int32, 16)
            rel = v - lo
            msk = (rel >= 0) & (rel < SPW)
            relc = jnp.where(msk, rel, 0)
            plsc.store_scatter(tok, [relc], tokval, mask=msk)
            plsc.store_scatter(gat, [relc], g, mask=msk)
            return 0

        lax.fori_loop(0, T // 16, body, 0)

    scatter_pass(t1_hbm, g1_hbm)
    scatter_pass(t2_hbm, g2_hbm)

    def gpass(j, _):
        pltpu.async_copy(x_hbm.at[tok.at[pl.ds(j * GCH, GCH)]], rows,
                         sem).wait()
        pltpu.sync_copy(rows, buf_hbm.at[pl.ds(lo + j * GCH, GCH)])
        return 0

    lax.fori_loop(0, SPW // GCH, gpass, 0)
    pltpu.sync_copy(gat, sg_hbm.at[pl.ds(lo, SPW)])


def _dispatch(x, t1, t2, g1, g2):
    mesh = plsc.VectorSubcoreMesh(core_axis_name="c", subcore_axis_name="s")
    return pl.kernel(
        _dispatch_body,
        out_type=[
            jax.ShapeDtypeStruct((E * C, D), jnp.float32),
            jax.ShapeDtypeStruct((E * C,), jnp.float32),
        ],
        mesh=mesh,
        scratch_types=[
            pltpu.VMEM((T,), jnp.int32),
            pltpu.VMEM((T,), jnp.float32),
            pltpu.VMEM((SPW,), jnp.int32),
            pltpu.VMEM((SPW,), jnp.float32),
            pltpu.VMEM((GCH, D), jnp.float32),
            pltpu.SemaphoreType.DMA,
        ],
        compiler_params=pltpu.CompilerParams(needs_layout_passes=False),
    )(x, t1, t2, g1, g2)


def _ffn_body(buf_ref, w1_ref, b1_ref, w2_ref, b2_ref, gate_ref, out_ref,
              acc_ref):
    f = pl.program_id(1)
    h = jnp.dot(buf_ref[0], w1_ref[0], preferred_element_type=jnp.float32)
    h = jnp.maximum(h + b1_ref[0], 0.0)
    partial = jnp.dot(h, w2_ref[0], preferred_element_type=jnp.float32)

    @pl.when(f == 0)
    def _first():
        acc_ref[...] = partial

    @pl.when(f > 0)
    def _rest():
        acc_ref[...] = acc_ref[...] + partial

    @pl.when(f == NF - 1)
    def _fin():
        out_ref[0] = (acc_ref[...] + b2_ref[0]) * gate_ref[0]


def _ffn(buf, w1, b1, w2, b2, gate):
    return pl.pallas_call(
        _ffn_body,
        grid=(E, NF),
        in_specs=[
            pl.BlockSpec((1, C, D), lambda e, f: (e, 0, 0)),
            pl.BlockSpec((1, D, FB), lambda e, f: (e, 0, f)),
            pl.BlockSpec((1, 1, FB), lambda e, f: (e, 0, f)),
            pl.BlockSpec((1, FB, D), lambda e, f: (e, f, 0)),
            pl.BlockSpec((1, 1, D), lambda e, f: (e, 0, 0)),
            pl.BlockSpec((1, C, 1), lambda e, f: (e, 0, 0)),
        ],
        out_specs=pl.BlockSpec((1, C, D), lambda e, f: (e, 0, 0)),
        out_shape=jax.ShapeDtypeStruct((E, C, D), jnp.float32),
        scratch_shapes=[pltpu.VMEM((C, D), jnp.float32)],
        compiler_params=pltpu.CompilerParams(
            dimension_semantics=("arbitrary", "arbitrary")),
    )(buf, w1, b1, w2, b2, gate)


def _combine_body(s_hbm, t1_hbm, t2_hbm, y_hbm, i1v, i2v, ra, rb, sem):
    wid = lax.axis_index("s") * 2 + lax.axis_index("c")
    base = wid * TPW
    pltpu.sync_copy(t1_hbm.at[pl.ds(base, TPW)], i1v)
    pltpu.sync_copy(t2_hbm.at[pl.ds(base, TPW)], i2v)

    def clamp(i, _):
        i1v[pl.ds(i * 16, 16)] = jnp.maximum(i1v[pl.ds(i * 16, 16)], 0)
        i2v[pl.ds(i * 16, 16)] = jnp.maximum(i2v[pl.ds(i * 16, 16)], 0)
        return 0

    lax.fori_loop(0, TPW // 16, clamp, 0)

    def chunk(j, _):
        pltpu.async_copy(s_hbm.at[i1v.at[pl.ds(j * GCH, GCH)]], ra, sem).wait()
        pltpu.async_copy(s_hbm.at[i2v.at[pl.ds(j * GCH, GCH)]], rb, sem).wait()

        def addrow(r_, _2):
            for cc in range(D // 16):
                sl = pl.ds(cc * 16, 16)
                ra[r_, sl] = ra[r_, sl] + rb[r_, sl]
            return 0

        lax.fori_loop(0, GCH, addrow, 0)
        pltpu.sync_copy(ra, y_hbm.at[pl.ds(base + j * GCH, GCH)])
        return 0

    lax.fori_loop(0, TPW // GCH, chunk, 0)


def _combine(scaled, t1, t2):
    mesh = plsc.VectorSubcoreMesh(core_axis_name="c", subcore_axis_name="s")
    return pl.kernel(
        _combine_body,
        out_type=jax.ShapeDtypeStruct((T, D), jnp.float32),
        mesh=mesh,
        scratch_types=[
            pltpu.VMEM((TPW,), jnp.int32),
            pltpu.VMEM((TPW,), jnp.int32),
            pltpu.VMEM((GCH, D), jnp.float32),
            pltpu.VMEM((GCH, D), jnp.float32),
            pltpu.SemaphoreType.DMA,
        ],
        compiler_params=pltpu.CompilerParams(needs_layout_passes=False),
    )(scaled, t1, t2)


def kernel(input, Wr, W1, b1, W2, b2):
    x = input
    t1, t2, g1, g2, loss = _router(x, Wr)
    t1 = t1.reshape(T)
    t2 = t2.reshape(T)
    g1 = g1.reshape(T)
    g2 = g2.reshape(T)
    buf, sg = _dispatch(x, t1, t2, g1, g2)
    out = _ffn(buf.reshape(E, C, D), W1, b1.reshape(E, 1, F), W2,
               b2.reshape(E, 1, D), sg.reshape(E, C, 1))
    y = _combine(out.reshape(E * C, D), t1, t2)
    return y, loss[0, 0]
